# Initial kernel scaffold; baseline (speedup 1.0000x reference)
import jax
import jax.numpy as jnp
from jax.experimental import pallas as pl


def kernel(image, pixel_W, pixel_b, edge_W, edge_b, edges):
    raise NotImplementedError("write your pallas kernel here")


# speedup vs baseline: 2.0319x; 2.0319x over previous
import functools

import jax
import jax.numpy as jnp
from jax import lax
from jax.experimental import pallas as pl
from jax.experimental.pallas import tpu as pltpu
from jax.experimental.pallas import tpu_sc as plsc

N_NODES = 50176
N_FEAT = 128
N_CLASSES = 21
N_EDGES = 99904
DPAD = 32


_M_BLK = 512


def _mm_body(x_ref, pw_ref, pb_ref, wa_ref, wb_ref, eb_ref,
             pix_ref, p_ref, q_ref):
    x = x_ref[...]
    pix_ref[...] = (jnp.dot(x, pw_ref[...], preferred_element_type=jnp.float32)
                    + pb_ref[...])
    p_ref[...] = (jnp.dot(x, wa_ref[...], preferred_element_type=jnp.float32)
                  + eb_ref[...])
    q_ref[...] = jnp.dot(x, wb_ref[...], preferred_element_type=jnp.float32)


def _mm_stage(x, pixel_W, pixel_b, wa_pad, wb_pad, eb_pad):
    grid = (N_NODES // _M_BLK,)
    return pl.pallas_call(
        _mm_body,
        grid=grid,
        in_specs=[
            pl.BlockSpec((_M_BLK, N_FEAT), lambda i: (i, 0)),
            pl.BlockSpec((N_FEAT, N_CLASSES), lambda i: (0, 0)),
            pl.BlockSpec((1, N_CLASSES), lambda i: (0, 0)),
            pl.BlockSpec((N_FEAT, DPAD), lambda i: (0, 0)),
            pl.BlockSpec((N_FEAT, DPAD), lambda i: (0, 0)),
            pl.BlockSpec((1, DPAD), lambda i: (0, 0)),
        ],
        out_specs=[
            pl.BlockSpec((_M_BLK, N_CLASSES), lambda i: (i, 0)),
            pl.BlockSpec((_M_BLK, DPAD), lambda i: (i, 0)),
            pl.BlockSpec((_M_BLK, DPAD), lambda i: (i, 0)),
        ],
        out_shape=[
            jax.ShapeDtypeStruct((N_NODES, N_CLASSES), jnp.float32),
            jax.ShapeDtypeStruct((N_NODES, DPAD), jnp.float32),
            jax.ShapeDtypeStruct((N_NODES, DPAD), jnp.float32),
        ],
    )(x, pixel_W, pixel_b, wa_pad, wb_pad, eb_pad)


_ROWS_PER_W = 3136
_CHUNK = 448


def _sc_edge_stage(p_tbl, q_tbl, src_idx, dst_idx):
    info = plsc.get_sparse_core_info()
    nc, ns = info.num_cores, info.num_subcores
    mesh = plsc.VectorSubcoreMesh(core_axis_name="c", subcore_axis_name="s")

    @functools.partial(
        pl.kernel,
        mesh=mesh,
        compiler_params=pltpu.CompilerParams(use_tc_tiling_on_sc=False),
        out_type=jax.ShapeDtypeStruct((N_EDGES, DPAD), jnp.float32),
        scratch_types=[
            pltpu.VMEM((_CHUNK,), jnp.int32),
            pltpu.VMEM((_CHUNK,), jnp.int32),
            pltpu.VMEM((_CHUNK, DPAD), jnp.float32),
            pltpu.VMEM((_CHUNK, DPAD), jnp.float32),
            pltpu.SemaphoreType.DMA,
            pltpu.SemaphoreType.DMA,
        ],
    )
    def sc_kernel(p_hbm, q_hbm, src_hbm, dst_hbm, out_hbm,
                  idx_s, idx_d, buf_p, buf_q, sem_p, sem_q):
        wid = lax.axis_index("s") * nc + lax.axis_index("c")
        base = wid * _ROWS_PER_W
        for i in range(_ROWS_PER_W // _CHUNK):
            row = base + i * _CHUNK

            @pl.when(row < N_EDGES)
            def _():
                pltpu.sync_copy(src_hbm.at[pl.ds(row, _CHUNK)], idx_s)
                pltpu.sync_copy(dst_hbm.at[pl.ds(row, _CHUNK)], idx_d)
                cp_p = pltpu.async_copy(p_hbm.at[idx_s], buf_p, sem_p)
                cp_q = pltpu.async_copy(q_hbm.at[idx_d], buf_q, sem_q)
                cp_p.wait()
                cp_q.wait()

                def add_row(r, carry):
                    buf_p[r, pl.ds(0, 16)] = buf_p[r, pl.ds(0, 16)] + buf_q[r, pl.ds(0, 16)]
                    buf_p[r, pl.ds(16, 16)] = buf_p[r, pl.ds(16, 16)] + buf_q[r, pl.ds(16, 16)]
                    return carry

                lax.fori_loop(0, _CHUNK, add_row, 0)
                pltpu.sync_copy(buf_p, out_hbm.at[pl.ds(row, _CHUNK), :])

    return sc_kernel(p_tbl, q_tbl, src_idx, dst_idx)


def kernel(image, pixel_W, pixel_b, edge_W, edge_b, edges):
    x = image.reshape(-1, N_FEAT)
    wa = edge_W[:N_FEAT]
    wb = edge_W[N_FEAT:]
    wa_pad = jnp.zeros((N_FEAT, DPAD), jnp.float32).at[:, :N_CLASSES].set(wa)
    wb_pad = jnp.zeros((N_FEAT, DPAD), jnp.float32).at[:, :N_CLASSES].set(wb)
    eb_pad = jnp.zeros((1, DPAD), jnp.float32).at[0, :N_CLASSES].set(edge_b)
    pb = pixel_b.reshape(1, N_CLASSES)

    pixel_pots, p_tbl, q_tbl = _mm_stage(x, pixel_W, pb, wa_pad, wb_pad, eb_pad)

    src_idx = edges[:, 0].astype(jnp.int32)
    dst_idx = edges[:, 1].astype(jnp.int32)
    edge_padded = _sc_edge_stage(p_tbl, q_tbl, src_idx, dst_idx)
    edge_pots = edge_padded[:, :N_CLASSES]
    return (pixel_pots, edge_pots)
